# baseline (speedup 1.0000x reference)
import functools

import jax
import jax.numpy as jnp
from jax import lax
from jax.experimental import pallas as pl
from jax.experimental.pallas import tpu as pltpu
from jax.experimental.pallas import tpu_sc as plsc

_B = 16384
_D = 64
_F = 64
_A = 10
_W = _D // 2

_NC, _NS = 2, 16
_NW = _NC * _NS
_BPW = _B // _NW
_HPW = _BPW // 2
_QPW = _BPW // 4


def _sc_gather(se, sa_g, obs, nobs, act):
    mesh = plsc.VectorSubcoreMesh(core_axis_name="c", subcore_axis_name="s")

    @functools.partial(
        pl.kernel,
        mesh=mesh,
        out_type=[
            jax.ShapeDtypeStruct((_B // 2, 2 * _D), jnp.float32),
            jax.ShapeDtypeStruct((_B // 4, 4 * _W), jnp.int32),
        ],
        scratch_types=[
            pltpu.VMEM((_BPW,), jnp.int32),
            pltpu.VMEM((_BPW,), jnp.int32),
            pltpu.VMEM((_BPW,), jnp.int32),
            pltpu.VMEM((_HPW, 2 * _D), jnp.float32),
            pltpu.VMEM((_QPW, 4 * _W), jnp.int32),
            pltpu.SemaphoreType.DMA,
        ],
    )
    def k(se_hbm, sa_hbm, obs_hbm, nobs_hbm, act_hbm,
          emb_out, sa_out,
          obs_v, nobs_v, act_v, emb_v, sa_v, sem):
        wid = lax.axis_index("s") * _NC + lax.axis_index("c")
        base = wid * _BPW
        pltpu.sync_copy(obs_hbm.at[pl.ds(base, _BPW)], obs_v)
        pltpu.sync_copy(nobs_hbm.at[pl.ds(base, _BPW)], nobs_v)
        pltpu.sync_copy(act_hbm.at[pl.ds(base, _BPW)], act_v)

        def make_fire(q):
            def fire(c, _):
                cb = c * 16
                src = q * _QPW + cb
                obs16 = obs_v[pl.ds(src, 16)]
                nobs16 = nobs_v[pl.ds(src, 16)]
                act16 = act_v[pl.ds(src, 16)]
                for j in range(16):
                    edst = ((q % 2) * _QPW + cb + j, pl.ds((q // 2) * _D, _D))
                    pltpu.async_copy(se_hbm.at[obs16[j]], emb_v.at[edst], sem)
                    sdst = (cb + j, pl.ds(q * _W, _W))
                    pltpu.async_copy(sa_hbm.at[act16[j], nobs16[j]],
                                     sa_v.at[sdst], sem)
                return 0
            return fire

        for q in range(4):
            lax.fori_loop(0, _QPW // 16, make_fire(q), 0)
        pltpu.make_async_copy(emb_out.at[pl.ds(0, _HPW)], emb_v, sem).wait()
        pltpu.make_async_copy(sa_out.at[pl.ds(0, _QPW)], sa_v, sem).wait()
        pltpu.sync_copy(emb_v, emb_out.at[pl.ds(wid * _HPW, _HPW)])
        pltpu.sync_copy(sa_v, sa_out.at[pl.ds(wid * _QPW, _QPW)])

    return k(se, sa_g, obs, nobs, act)


_COS_C = (
    0.9999999999999938, -19.73920880217503, 64.93939402216306,
    -85.45681717974715, 60.24464064338281, -26.42624548946228,
    7.903429882766466, -1.7137692085152525, 0.27980881692562937,
    -0.032045487143534404,
)
_INV_2PI = 0.15915494309189535


def _fast_cos(x):
    r = x * _INV_2PI
    r = r - jnp.round(r)
    u = r * r
    p = jnp.full_like(u, _COS_C[-1])
    for c in _COS_C[-2::-1]:
        p = p * u + c
    return p


def _tc_body(emb_ref, sa_ref, act_ref, om_ref, om2_ref, sh_ref, ae_ref,
             aq_ref, pol_ref, out_ref):
    ep = emb_ref[...]
    sp = sa_ref[...]
    x = jnp.concatenate([ep[:, :_D], ep[:, _D:]], axis=0)
    sw = jnp.concatenate([sp[:, :_W], sp[:, _W:2 * _W],
                          sp[:, 2 * _W:3 * _W], sp[:, 3 * _W:]],
                         axis=0)
    lo = lax.bitcast_convert_type(lax.shift_left(sw, 16), jnp.float32)
    hi = lax.bitcast_convert_type(
        lax.bitwise_and(sw, jnp.int32(-65536)), jnp.float32)
    sa = jnp.concatenate([lo, hi], axis=1)
    ae = ae_ref[...]
    std = jnp.sqrt(jnp.maximum(1e-8, aq_ref[...] - ae * ae))
    x = (x - ae) / std
    om = om_ref[...]
    proj = lax.dot_general(x, om, (((1,), (1,)), ((), ())),
                           preferred_element_type=jnp.float32)
    proj = proj * (1.0 / (_D ** 0.5))
    el = _fast_cos(proj + sh_ref[...])
    logits = jnp.dot(el, pol_ref[...], preferred_element_type=jnp.float32)
    m = jnp.max(logits, axis=1, keepdims=True)
    e = jnp.exp(logits - m)
    probs = e / jnp.sum(e, axis=1, keepdims=True)
    ne = _fast_cos(lax.dot_general(sa, om2_ref[...], (((1,), (1,)), ((), ())),
                                   preferred_element_type=jnp.float32)
                   + sh_ref[...])
    et = jnp.sum(el * ne, axis=1, keepdims=True) * ((2.0 / _F) ** 0.5)
    iota = lax.broadcasted_iota(jnp.int32, (_BPW, _A), 1)
    ap = jnp.sum(jnp.where(iota == act_ref[...], probs, 0.0),
                 axis=1, keepdims=True)
    out_ref[...] = jnp.concatenate([probs, ap, et], axis=1)


def _tc_dense(emb_p, sa_p, act2, omega, omega2, shift2, ae2, aq2, policy):
    return pl.pallas_call(
        _tc_body,
        grid=(_NW,),
        in_specs=[
            pl.BlockSpec((_HPW, 2 * _D), lambda i: (i, 0)),
            pl.BlockSpec((_QPW, 4 * _W), lambda i: (i, 0)),
            pl.BlockSpec((_BPW, 1), lambda i: (i, 0)),
            pl.BlockSpec((_F, _D), lambda i: (0, 0)),
            pl.BlockSpec((_F, _D), lambda i: (0, 0)),
            pl.BlockSpec((1, _F), lambda i: (0, 0)),
            pl.BlockSpec((1, _D), lambda i: (0, 0)),
            pl.BlockSpec((1, _D), lambda i: (0, 0)),
            pl.BlockSpec((_F, _A), lambda i: (0, 0)),
        ],
        out_specs=pl.BlockSpec((_BPW, _A + 2), lambda i: (i, 0)),
        out_shape=jax.ShapeDtypeStruct((_B, _A + 2), jnp.float32),
    )(emb_p, sa_p, act2, omega, omega2, shift2, ae2, aq2, policy)


def kernel(observation, action, next_observation, state_embedder,
           state_action_embedder, omega, shift, average_embed,
           average_square, embed_policy):
    sa_g = jnp.transpose(state_action_embedder, (1, 0, 2))
    sa_bf = sa_g.astype(jnp.bfloat16).reshape(_A, -1, _W, 2)
    sa_w = lax.bitcast_convert_type(sa_bf, jnp.int32)
    omega2 = jnp.concatenate([omega[:, ::2], omega[:, 1::2]], axis=1)
    emb_p, sa_p = _sc_gather(state_embedder, sa_w, observation,
                             next_observation, action)
    return _tc_dense(
        emb_p, sa_p,
        action.reshape(_B, 1),
        omega,
        omega2,
        shift.reshape(1, _F),
        average_embed.reshape(1, _D),
        average_square.reshape(1, _D),
        embed_policy,
    )


# speedup vs baseline: 2.0278x; 2.0278x over previous
import functools

import jax
import jax.numpy as jnp
from jax import lax
from jax.experimental import pallas as pl
from jax.experimental.pallas import tpu as pltpu
from jax.experimental.pallas import tpu_sc as plsc

_B = 16384
_D = 64
_F = 64
_A = 10
_S = 100000
_AH = _A // 2

_NC, _NS = 2, 16
_NW = _NC * _NS
_BPW = _B // _NW
_HPW = _BPW // 2

_SBLK = 2048


def _tc_relayout(x_t, eye):
    g, _, s = x_t.shape

    def body(x_ref, eye_ref, o_ref):
        o_ref[0] = lax.dot_general(x_ref[0], eye_ref[...],
                                   (((0,), (0,)), ((), ())),
                                   preferred_element_type=jnp.float32)

    return pl.pallas_call(
        body,
        grid=(g, pl.cdiv(s, _SBLK)),
        in_specs=[
            pl.BlockSpec((1, _D, _SBLK), lambda i, j: (i, 0, j)),
            pl.BlockSpec((_D, _D), lambda i, j: (0, 0)),
        ],
        out_specs=pl.BlockSpec((1, _SBLK, _D), lambda i, j: (i, j, 0)),
        out_shape=jax.ShapeDtypeStruct((g, s, _D), jnp.float32),
    )(x_t, eye)


def _sc_gather(se, sa_lo, sa_hi, obs, nobs, act):
    mesh = plsc.VectorSubcoreMesh(core_axis_name="c", subcore_axis_name="s")

    @functools.partial(
        pl.kernel,
        mesh=mesh,
        out_type=[
            jax.ShapeDtypeStruct((_B // 2, 2 * _D), jnp.float32),
            jax.ShapeDtypeStruct((_B // 2, 2 * _D), jnp.float32),
        ],
        scratch_types=[
            pltpu.VMEM((_BPW,), jnp.int32),
            pltpu.VMEM((_BPW,), jnp.int32),
            pltpu.VMEM((_BPW,), jnp.int32),
            pltpu.VMEM((_HPW, 2 * _D), jnp.float32),
            pltpu.VMEM((_HPW, 2 * _D), jnp.float32),
            pltpu.SemaphoreType.DMA,
        ],
    )
    def k(se_hbm, salo_hbm, sahi_hbm, obs_hbm, nobs_hbm, act_hbm,
          emb_out, sa_out,
          obs_v, nobs_v, act_v, emb_v, sa_v, sem):
        wid = lax.axis_index("s") * _NC + lax.axis_index("c")
        base = wid * _BPW
        pltpu.sync_copy(obs_hbm.at[pl.ds(base, _BPW)], obs_v)
        pltpu.sync_copy(nobs_hbm.at[pl.ds(base, _BPW)], nobs_v)
        pltpu.sync_copy(act_hbm.at[pl.ds(base, _BPW)], act_v)

        def make_fire(off):
            def fire(c, _):
                cb = c * 16
                src = off * _HPW + cb
                obs16 = obs_v[pl.ds(src, 16)]
                nobs16 = nobs_v[pl.ds(src, 16)]
                act16 = act_v[pl.ds(src, 16)]
                for j in range(16):
                    dst = (cb + j, pl.ds(off * _D, _D))
                    pltpu.async_copy(se_hbm.at[obs16[j]], emb_v.at[dst], sem)
                    a = act16[j]
                    ns = nobs16[j]

                    @pl.when(a < _AH)
                    def _():
                        pltpu.async_copy(salo_hbm.at[a, ns],
                                         sa_v.at[dst], sem)

                    @pl.when(a >= _AH)
                    def _():
                        pltpu.async_copy(sahi_hbm.at[a - _AH, ns],
                                         sa_v.at[dst], sem)
                return 0
            return fire

        lax.fori_loop(0, _HPW // 16, make_fire(0), 0)
        lax.fori_loop(0, _HPW // 16, make_fire(1), 0)
        pltpu.make_async_copy(emb_out.at[pl.ds(0, _HPW)], emb_v, sem).wait()
        pltpu.make_async_copy(emb_out.at[pl.ds(0, _HPW)], sa_v, sem).wait()
        pltpu.sync_copy(emb_v, emb_out.at[pl.ds(wid * _HPW, _HPW)])
        pltpu.sync_copy(sa_v, sa_out.at[pl.ds(wid * _HPW, _HPW)])

    return k(se, sa_lo, sa_hi, obs, nobs, act)


_COS_C = (
    0.9999999999999938, -19.73920880217503, 64.93939402216306,
    -85.45681717974715, 60.24464064338281, -26.42624548946228,
    7.903429882766466, -1.7137692085152525, 0.27980881692562937,
    -0.032045487143534404,
)
_INV_2PI = 0.15915494309189535


def _fast_cos(x):
    r = x * _INV_2PI
    r = r - jnp.round(r)
    u = r * r
    p = jnp.full_like(u, _COS_C[-1])
    for c in _COS_C[-2::-1]:
        p = p * u + c
    return p


def _tc_body(emb_ref, sa_ref, act_ref, om_ref, sh_ref, ae_ref, aq_ref, pol_ref,
             out_ref):
    ep = emb_ref[...]
    sp = sa_ref[...]
    x = jnp.concatenate([ep[:, :_D], ep[:, _D:]], axis=0)
    sa = jnp.concatenate([sp[:, :_D], sp[:, _D:]], axis=0)
    ae = ae_ref[...]
    std = jnp.sqrt(jnp.maximum(1e-8, aq_ref[...] - ae * ae))
    x = (x - ae) / std
    om = om_ref[...]
    proj = lax.dot_general(x, om, (((1,), (1,)), ((), ())),
                           preferred_element_type=jnp.float32)
    proj = proj * (1.0 / (_D ** 0.5))
    el = _fast_cos(proj + sh_ref[...])
    logits = jnp.dot(el, pol_ref[...], preferred_element_type=jnp.float32)
    m = jnp.max(logits, axis=1, keepdims=True)
    e = jnp.exp(logits - m)
    probs = e / jnp.sum(e, axis=1, keepdims=True)
    ne = _fast_cos(lax.dot_general(sa, om, (((1,), (1,)), ((), ())),
                                   preferred_element_type=jnp.float32)
                   + sh_ref[...])
    et = jnp.sum(el * ne, axis=1, keepdims=True) * ((2.0 / _F) ** 0.5)
    iota = lax.broadcasted_iota(jnp.int32, (_BPW, _A), 1)
    ap = jnp.sum(jnp.where(iota == act_ref[...], probs, 0.0),
                 axis=1, keepdims=True)
    out_ref[...] = jnp.concatenate([probs, ap, et], axis=1)


def _tc_dense(emb_p, sa_p, act2, omega, shift2, ae2, aq2, policy):
    return pl.pallas_call(
        _tc_body,
        grid=(_NW,),
        in_specs=[
            pl.BlockSpec((_HPW, 2 * _D), lambda i: (i, 0)),
            pl.BlockSpec((_HPW, 2 * _D), lambda i: (i, 0)),
            pl.BlockSpec((_BPW, 1), lambda i: (i, 0)),
            pl.BlockSpec((_F, _D), lambda i: (0, 0)),
            pl.BlockSpec((1, _F), lambda i: (0, 0)),
            pl.BlockSpec((1, _D), lambda i: (0, 0)),
            pl.BlockSpec((1, _D), lambda i: (0, 0)),
            pl.BlockSpec((_F, _A), lambda i: (0, 0)),
        ],
        out_specs=pl.BlockSpec((_BPW, _A + 2), lambda i: (i, 0)),
        out_shape=jax.ShapeDtypeStruct((_B, _A + 2), jnp.float32),
    )(emb_p, sa_p, act2, omega, shift2, ae2, aq2, policy)


def kernel(observation, action, next_observation, state_embedder,
           state_action_embedder, omega, shift, average_embed,
           average_square, embed_policy):
    eye = jnp.eye(_D, dtype=jnp.float32)
    sa_lo = jnp.transpose(state_action_embedder[:, :_AH], (1, 0, 2))
    sa_hi_t = jnp.transpose(state_action_embedder[:, _AH:], (1, 2, 0))
    sa_hi = _tc_relayout(sa_hi_t, eye)
    se_rm = _tc_relayout(state_embedder.T[None], eye)[0]
    emb_p, sa_p = _sc_gather(se_rm, sa_lo, sa_hi, observation,
                             next_observation, action)
    return _tc_dense(
        emb_p, sa_p,
        action.reshape(_B, 1),
        omega,
        shift.reshape(1, _F),
        average_embed.reshape(1, _D),
        average_square.reshape(1, _D),
        embed_policy,
    )


# speedup vs baseline: 4.9828x; 2.4573x over previous
import functools

import jax
import jax.numpy as jnp
from jax import lax
from jax.experimental import pallas as pl
from jax.experimental.pallas import tpu as pltpu
from jax.experimental.pallas import tpu_sc as plsc

_B = 16384
_D = 64
_F = 64
_A = 10

_NC, _NS = 2, 16
_NW = _NC * _NS
_BPW = _B // _NW
_HPW = _BPW // 2


def _sc_gather(se, sa_g, obs, nobs, act):
    mesh = plsc.VectorSubcoreMesh(core_axis_name="c", subcore_axis_name="s")

    @functools.partial(
        pl.kernel,
        mesh=mesh,
        out_type=[
            jax.ShapeDtypeStruct((_B // 2, 2 * _D), jnp.float32),
            jax.ShapeDtypeStruct((_B // 2, 2 * _D), jnp.float32),
        ],
        scratch_types=[
            pltpu.VMEM((_BPW,), jnp.int32),
            pltpu.VMEM((_BPW,), jnp.int32),
            pltpu.VMEM((_BPW,), jnp.int32),
            pltpu.VMEM((_HPW, 2 * _D), jnp.float32),
            pltpu.VMEM((_HPW, 2 * _D), jnp.float32),
            pltpu.SemaphoreType.DMA,
        ],
    )
    def k(se_hbm, sa_hbm, obs_hbm, nobs_hbm, act_hbm,
          emb_out, sa_out,
          obs_v, nobs_v, act_v, emb_v, sa_v, sem):
        wid = lax.axis_index("s") * _NC + lax.axis_index("c")
        base = wid * _BPW
        pltpu.sync_copy(obs_hbm.at[pl.ds(base, _BPW)], obs_v)
        pltpu.sync_copy(nobs_hbm.at[pl.ds(base, _BPW)], nobs_v)
        pltpu.sync_copy(act_hbm.at[pl.ds(base, _BPW)], act_v)

        def make_fire(off):
            def fire(c, _):
                cb = c * 16
                src = off * _HPW + cb
                obs16 = obs_v[pl.ds(src, 16)]
                nobs16 = nobs_v[pl.ds(src, 16)]
                act16 = act_v[pl.ds(src, 16)]
                for j in range(16):
                    dst = (cb + j, pl.ds(off * _D, _D))
                    pltpu.async_copy(se_hbm.at[obs16[j]], emb_v.at[dst], sem)
                    pltpu.async_copy(sa_hbm.at[act16[j], nobs16[j]],
                                     sa_v.at[dst], sem)
                return 0
            return fire

        lax.fori_loop(0, _HPW // 16, make_fire(0), 0)
        lax.fori_loop(0, _HPW // 16, make_fire(1), 0)
        pltpu.make_async_copy(emb_out.at[pl.ds(0, _HPW)], emb_v, sem).wait()
        pltpu.make_async_copy(emb_out.at[pl.ds(0, _HPW)], sa_v, sem).wait()
        pltpu.sync_copy(emb_v, emb_out.at[pl.ds(wid * _HPW, _HPW)])
        pltpu.sync_copy(sa_v, sa_out.at[pl.ds(wid * _HPW, _HPW)])

    return k(se, sa_g, obs, nobs, act)


_COS_C = (
    0.9999999999999938, -19.73920880217503, 64.93939402216306,
    -85.45681717974715, 60.24464064338281, -26.42624548946228,
    7.903429882766466, -1.7137692085152525, 0.27980881692562937,
    -0.032045487143534404,
)
_INV_2PI = 0.15915494309189535


def _fast_cos(x):
    r = x * _INV_2PI
    r = r - jnp.round(r)
    u = r * r
    p = jnp.full_like(u, _COS_C[-1])
    for c in _COS_C[-2::-1]:
        p = p * u + c
    return p


def _tc_body(emb_ref, sa_ref, act_ref, om_ref, sh_ref, ae_ref, aq_ref, pol_ref,
             out_ref):
    ep = emb_ref[...]
    sp = sa_ref[...]
    x = jnp.concatenate([ep[:, :_D], ep[:, _D:]], axis=0)
    sa = jnp.concatenate([sp[:, :_D], sp[:, _D:]], axis=0)
    ae = ae_ref[...]
    std = jnp.sqrt(jnp.maximum(1e-8, aq_ref[...] - ae * ae))
    x = (x - ae) / std
    om = om_ref[...]
    proj = lax.dot_general(x, om, (((1,), (1,)), ((), ())),
                           preferred_element_type=jnp.float32)
    proj = proj * (1.0 / (_D ** 0.5))
    el = _fast_cos(proj + sh_ref[...])
    logits = jnp.dot(el, pol_ref[...], preferred_element_type=jnp.float32)
    m = jnp.max(logits, axis=1, keepdims=True)
    e = jnp.exp(logits - m)
    probs = e / jnp.sum(e, axis=1, keepdims=True)
    ne = _fast_cos(lax.dot_general(sa, om, (((1,), (1,)), ((), ())),
                                   preferred_element_type=jnp.float32)
                   + sh_ref[...])
    et = jnp.sum(el * ne, axis=1, keepdims=True) * ((2.0 / _F) ** 0.5)
    iota = lax.broadcasted_iota(jnp.int32, (_BPW, _A), 1)
    ap = jnp.sum(jnp.where(iota == act_ref[...], probs, 0.0),
                 axis=1, keepdims=True)
    out_ref[...] = jnp.concatenate([probs, ap, et], axis=1)


def _tc_dense(emb_p, sa_p, act2, omega, shift2, ae2, aq2, policy):
    return pl.pallas_call(
        _tc_body,
        grid=(_NW,),
        in_specs=[
            pl.BlockSpec((_HPW, 2 * _D), lambda i: (i, 0)),
            pl.BlockSpec((_HPW, 2 * _D), lambda i: (i, 0)),
            pl.BlockSpec((_BPW, 1), lambda i: (i, 0)),
            pl.BlockSpec((_F, _D), lambda i: (0, 0)),
            pl.BlockSpec((1, _F), lambda i: (0, 0)),
            pl.BlockSpec((1, _D), lambda i: (0, 0)),
            pl.BlockSpec((1, _D), lambda i: (0, 0)),
            pl.BlockSpec((_F, _A), lambda i: (0, 0)),
        ],
        out_specs=pl.BlockSpec((_BPW, _A + 2), lambda i: (i, 0)),
        out_shape=jax.ShapeDtypeStruct((_B, _A + 2), jnp.float32),
    )(emb_p, sa_p, act2, omega, shift2, ae2, aq2, policy)


def kernel(observation, action, next_observation, state_embedder,
           state_action_embedder, omega, shift, average_embed,
           average_square, embed_policy):
    sa_g = jnp.transpose(state_action_embedder, (1, 0, 2))
    emb_p, sa_p = _sc_gather(state_embedder, sa_g, observation,
                             next_observation, action)
    return _tc_dense(
        emb_p, sa_p,
        action.reshape(_B, 1),
        omega,
        shift.reshape(1, _F),
        average_embed.reshape(1, _D),
        average_square.reshape(1, _D),
        embed_policy,
    )
